# Initial kernel scaffold; baseline (speedup 1.0000x reference)
import jax
import jax.numpy as jnp
from jax.experimental import pallas as pl


def kernel(x, edge_index, weight, root_weight, bias):
    raise NotImplementedError("write your pallas kernel here")


# speedup vs baseline: 5.1121x; 5.1121x over previous
import functools

import jax
import jax.numpy as jnp
from jax import lax
from jax.experimental import pallas as pl
from jax.experimental.pallas import tpu as pltpu
from jax.experimental.pallas import tpu_sc as plsc

NC = 2
NS = 16
LANES = 16
CHUNK = 128
HALF = 128


def _mm_body(x_ref, w_ref, b_ref, tlo_ref, thi_ref, root_ref):
    o = jnp.dot(x_ref[...], w_ref[...], preferred_element_type=jnp.float32)
    d = tlo_ref.shape[1]
    tlo_ref[...] = o[:, :d]
    thi_ref[...] = o[:, d:2 * d]
    root_ref[...] = o[:, 2 * d:] + b_ref[...]


def _matmul(x, wcat, bias_row, bn):
    n, d_in = x.shape
    d_out = bias_row.shape[1]
    grid = n // bn
    return pl.pallas_call(
        _mm_body,
        grid=(grid,),
        in_specs=[
            pl.BlockSpec((bn, d_in), lambda i: (i, 0)),
            pl.BlockSpec((d_in, 2 * d_out), lambda i: (0, 0)),
            pl.BlockSpec((1, d_out), lambda i: (0, 0)),
        ],
        out_specs=[
            pl.BlockSpec((bn, HALF), lambda i: (i, 0)),
            pl.BlockSpec((bn, HALF), lambda i: (i, 0)),
            pl.BlockSpec((bn, d_out), lambda i: (i, 0)),
        ],
        out_shape=[
            jax.ShapeDtypeStruct((n, HALF), jnp.float32),
            jax.ShapeDtypeStruct((n, HALF), jnp.float32),
            jax.ShapeDtypeStruct((n, d_out), jnp.float32),
        ],
    )(x, wcat, bias_row)


def _sc_aggregate(tlo, thi, edge_index, n_nodes, n_edges):
    n_pad = ((n_nodes + NS * CHUNK - 1) // (NS * CHUNK)) * (NS * CHUNK)
    rows_per_tile = n_pad // NS
    num_chunks = n_edges // CHUNK

    mesh = plsc.VectorSubcoreMesh(core_axis_name="c", subcore_axis_name="s")

    @functools.partial(
        pl.kernel,
        mesh=mesh,
        out_type=(
            jax.ShapeDtypeStruct((NC, n_pad, HALF), jnp.float32),
            jax.ShapeDtypeStruct((n_pad,), jnp.float32),
        ),
        scratch_types=[
            pltpu.VMEM_SHARED((n_pad, HALF), jnp.float32),
            pltpu.VMEM_SHARED((n_pad,), jnp.float32),
            pltpu.VMEM((1, CHUNK), jnp.int32),
            pltpu.VMEM((1, CHUNK), jnp.int32),
            pltpu.VMEM((CHUNK, HALF), jnp.float32),
            pltpu.VMEM((CHUNK,), jnp.float32),
            pltpu.SemaphoreType.DMA,
        ],
    )
    def agg_kernel(tlo_hbm, thi_hbm, edges_hbm, agg_hbm, deg_hbm,
                   agg_s, deg_s, ridx, cidx, msgs, ones, sem):
        c = lax.axis_index("c")
        t = lax.axis_index("s")
        r0 = t * rows_per_tile

        def zrow(r, _):
            def zcol(j, _):
                msgs[r, pl.ds(j * LANES, LANES)] = jnp.zeros(
                    (LANES,), jnp.float32)
                return 0
            return lax.fori_loop(0, HALF // LANES, zcol, 0)
        lax.fori_loop(0, CHUNK, zrow, 0)

        def zon(j, _):
            ones[pl.ds(j * LANES, LANES)] = jnp.zeros((LANES,), jnp.float32)
            return 0
        lax.fori_loop(0, CHUNK // LANES, zon, 0)

        for b in range(rows_per_tile // CHUNK):
            pltpu.sync_copy(msgs, agg_s.at[pl.ds(r0 + b * CHUNK, CHUNK)])
            pltpu.sync_copy(ones, deg_s.at[pl.ds(r0 + b * CHUNK, CHUNK)])

        def son(j, _):
            ones[pl.ds(j * LANES, LANES)] = jnp.ones((LANES,), jnp.float32)
            return 0
        lax.fori_loop(0, CHUNK // LANES, son, 0)

        plsc.subcore_barrier()

        nk = (num_chunks - t + NS - 1) // NS

        def ebody(k, _):
            base = (t + k * NS) * CHUNK
            pltpu.sync_copy(edges_hbm.at[0, pl.ds(base, CHUNK)], ridx.at[0])
            pltpu.sync_copy(edges_hbm.at[1, pl.ds(base, CHUNK)], cidx.at[0])

            @pl.when(c == 0)
            def _():
                pltpu.async_copy(tlo_hbm.at[cidx.at[0]], msgs, sem).wait()

            @pl.when(c == 1)
            def _():
                pltpu.async_copy(thi_hbm.at[cidx.at[0]], msgs, sem).wait()

            pltpu.sync_copy(msgs, agg_s.at[ridx.at[0]], add=True)
            pltpu.sync_copy(ones, deg_s.at[ridx.at[0]], add=True)
            return 0
        lax.fori_loop(0, nk, ebody, 0)

        plsc.subcore_barrier()

        pltpu.sync_copy(agg_s.at[pl.ds(r0, rows_per_tile)],
                        agg_hbm.at[c, pl.ds(r0, rows_per_tile)])

        @pl.when(c == 0)
        def _():
            pltpu.sync_copy(deg_s.at[pl.ds(r0, rows_per_tile)],
                            deg_hbm.at[pl.ds(r0, rows_per_tile)])

    return agg_kernel(tlo, thi, edge_index)


def _fin_body(agg_ref, deg_ref, root_ref, out_ref):
    d = jnp.maximum(deg_ref[...], 1.0)
    a = jnp.concatenate([agg_ref[0], agg_ref[1]], axis=-1)
    out_ref[...] = a / d + root_ref[...]


def _finalize(agg, deg_col, root, bn):
    n, d_out = root.shape
    grid = n // bn
    return pl.pallas_call(
        _fin_body,
        grid=(grid,),
        in_specs=[
            pl.BlockSpec((NC, bn, HALF), lambda i: (0, i, 0)),
            pl.BlockSpec((bn, 1), lambda i: (i, 0)),
            pl.BlockSpec((bn, d_out), lambda i: (i, 0)),
        ],
        out_specs=pl.BlockSpec((bn, d_out), lambda i: (i, 0)),
        out_shape=jax.ShapeDtypeStruct((n, d_out), jnp.float32),
    )(agg, deg_col, root)


def kernel(x, edge_index, weight, root_weight, bias):
    n, _ = x.shape
    e = edge_index.shape[1]
    wcat = jnp.concatenate([weight, root_weight], axis=1)
    tlo, thi, root = _matmul(x, wcat, bias.reshape(1, -1), bn=1000)
    agg, deg = _sc_aggregate(tlo, thi, edge_index, n, e)
    return _finalize(agg, deg.reshape(-1, 1), root, bn=1000)
